# baseline (speedup 1.0000x reference)
import functools

import jax
import jax.numpy as jnp
from jax import lax
from jax.experimental import pallas as pl
from jax.experimental.pallas import tpu as pltpu
from jax.experimental.pallas import tpu_sc as plsc

_N = 10000
_D = 128
_C = 64
_HC = _C // 2
_E = 320000
_NC = 2
_NS = 16
_NW = _NC * _NS
_CH = 128
_CHD = 80
_CHH = 160
_EPAD = _NW * _CHD * _CH
_NPAD = 10112
_ZS = _NPAD // _NS
_NB = 4
_PD = 2
_LASTY = 15 * _ZS
_LASTN = _N - _LASTY


def _vmesh():
    return plsc.VectorSubcoreMesh(core_axis_name="c", subcore_axis_name="s")


def _newton_rsqrt(v):
    i = lax.bitcast_convert_type(v, jnp.int32)
    i = 0x5F3759DF - lax.shift_right_logical(i, 1)
    g = lax.bitcast_convert_type(i, jnp.float32)
    h = v * 0.5
    for _ in range(3):
        g = g * (1.5 - h * g * g)
    return g


def _sc_degree(col_t, z16, ones16):
    @functools.partial(
        pl.kernel,
        out_type=jax.ShapeDtypeStruct((_NC, _NPAD, 16), jnp.float32),
        mesh=_vmesh(),
        compiler_params=pltpu.CompilerParams(use_tc_tiling_on_sc=False),
        scratch_types=[
            pltpu.VMEM((_CHD, _CH), jnp.int32),
            pltpu.VMEM((_CH, 16), jnp.float32),
            pltpu.VMEM_SHARED((_NPAD, 16), jnp.float32),
        ],
    )
    def k(col_ref, z_ref, ones_ref, out_ref, col_v, ones_v, acc):
        cid = lax.axis_index("c")
        sid = lax.axis_index("s")
        wid = sid * _NC + cid
        pltpu.sync_copy(z_ref, acc.at[pl.ds(sid * _ZS, _ZS)])
        pltpu.sync_copy(col_ref.at[wid], col_v)
        pltpu.sync_copy(ones_ref, ones_v)
        plsc.subcore_barrier()

        @pl.loop(0, _CHD)
        def _(j):
            pltpu.sync_copy(ones_v, acc.at[col_v.at[j]], add=True)

        plsc.subcore_barrier()
        pltpu.sync_copy(acc.at[pl.ds(sid * _ZS, _ZS)],
                        out_ref.at[cid, pl.ds(sid * _ZS, _ZS)])

    return k(col_t, z16, ones16)


def _sc_hops(xwa, xwb, degp, row_t, col_t, zpad, b4):
    @functools.partial(
        pl.kernel,
        out_type=jax.ShapeDtypeStruct((_N, _C), jnp.float32),
        mesh=_vmesh(),
        compiler_params=pltpu.CompilerParams(use_tc_tiling_on_sc=False),
        scratch_types=[
            pltpu.VMEM((_CHH, _CH), jnp.int32),
            pltpu.VMEM((_CHH, _CH), jnp.int32),
            [pltpu.VMEM((_CH, _HC), jnp.float32) for _ in range(_NB)],
            pltpu.VMEM((_ZS, _HC), jnp.float32),
            pltpu.VMEM((_ZS, 16), jnp.float32),
            pltpu.VMEM((2, 16), jnp.float32),
            pltpu.SemaphoreType.DMA((_NB,)),
            pltpu.SemaphoreType.DMA((_NB,)),
            pltpu.VMEM_SHARED((_NPAD, _HC), jnp.float32),
            pltpu.VMEM_SHARED((_N, _HC), jnp.float32),
        ],
    )
    def k(xwa_ref, xwb_ref, dp_ref, row_ref, col_ref, zpad_ref, b_ref, o_ref,
          row_v, col_v, gb, ybuf, dv, bbuf, semg, sems, acc, ysh):
        cid = lax.axis_index("c")
        sid = lax.axis_index("s")
        pltpu.sync_copy(row_ref.at[sid], row_v)
        pltpu.sync_copy(col_ref.at[sid], col_v)
        pltpu.sync_copy(b_ref.at[pl.ds(cid * 2, 2)], bbuf)

        sl = pl.ds(sid * _ZS, _ZS)
        pltpu.sync_copy(dp_ref.at[0, sl], dv)
        pltpu.sync_copy(dp_ref.at[1, sl], ybuf.at[:, pl.ds(0, 16)])

        @pl.loop(0, _ZS // 8)
        def _(rb):
            for k in range(8):
                r = rb * 8 + k
                dv[r] = _newton_rsqrt(dv[r] + ybuf[r, pl.ds(0, 16)] + 1.0)

        def scale_rows(n):
            @pl.loop(0, n // 8)
            def _(rb):
                for k in range(8):
                    r = rb * 8 + k
                    s = dv[r]
                    for q in (0, 16):
                        ybuf[r, pl.ds(q, 16)] = ybuf[r, pl.ds(q, 16)] * s

        def stage(xw_ref):
            @pl.when(sid < _NS - 1)
            def _():
                pltpu.sync_copy(xw_ref.at[sl], ybuf)
                scale_rows(_ZS)
                pltpu.sync_copy(ybuf, ysh.at[sl])
                pltpu.sync_copy(ybuf, acc.at[sl])

            @pl.when(sid == _NS - 1)
            def _():
                ll = pl.ds(_LASTY, _LASTN)
                bl = pl.ds(0, _LASTN)
                pltpu.sync_copy(xw_ref.at[ll], ybuf.at[bl])
                scale_rows(_LASTN)
                pltpu.sync_copy(ybuf.at[bl], ysh.at[ll])
                pltpu.sync_copy(ybuf.at[bl], acc.at[ll])
                pltpu.sync_copy(zpad_ref, acc.at[pl.ds(_N, _NPAD - _N)])

        @pl.when(cid == 0)
        def _():
            stage(xwa_ref)

        @pl.when(cid == 1)
        def _():
            stage(xwb_ref)

        plsc.subcore_barrier()

        def gather(c, p):
            pltpu.async_copy(ysh.at[row_v.at[c]], gb[p], semg.at[p])

        def wait_gather(c, p):
            pltpu.make_async_copy(ysh.at[row_v.at[c]], gb[p],
                                  semg.at[p]).wait()

        def scatter(c, p):
            pltpu.async_copy(gb[p], acc.at[col_v.at[c]], sems.at[p], add=True)

        def wait_scatter(c, p):
            pltpu.make_async_copy(gb[p], acc.at[col_v.at[c]],
                                  sems.at[p]).wait()

        def hop_loop():
            for p in range(_PD):
                gather(p, p)

            @pl.loop(0, _CHH // _NB)
            def _(t):
                j0 = t * _NB
                for p in range(_NB):
                    j = j0 + p
                    pn = (p + _PD) % _NB

                    @pl.when(j + _PD >= _NB)
                    def _():
                        wait_scatter(j + _PD - _NB, pn)

                    @pl.when(j + _PD < _CHH)
                    def _():
                        gather(j + _PD, pn)

                    wait_gather(j, p)
                    scatter(j, p)

            for c in range(_CHH - _NB, _CHH):
                if c + _NB - _PD > _CHH - 1:
                    wait_scatter(c, c % _NB)

            plsc.subcore_barrier()

        hop_loop()

        pltpu.sync_copy(acc.at[sl], ybuf)

        @pl.loop(0, _ZS // 8)
        def _(rb):
            for k in range(8):
                r = rb * 8 + k
                s = dv[r]
                s2 = s * s
                for q in (0, 16):
                    ybuf[r, pl.ds(q, 16)] = ybuf[r, pl.ds(q, 16)] * s2

        pltpu.sync_copy(ybuf, acc.at[sl])

        @pl.when(sid < _NS - 1)
        def _():
            pltpu.sync_copy(ybuf, ysh.at[sl])

        @pl.when(sid == _NS - 1)
        def _():
            pltpu.sync_copy(ybuf.at[pl.ds(0, _LASTN)],
                            ysh.at[pl.ds(_LASTY, _LASTN)])

        plsc.subcore_barrier()

        hop_loop()

        pltpu.sync_copy(acc.at[sl], ybuf)

        @pl.loop(0, _ZS // 8)
        def _(rb):
            for k in range(8):
                r = rb * 8 + k
                s = dv[r]
                for qi in (0, 1):
                    q = qi * 16
                    ybuf[r, pl.ds(q, 16)] = (ybuf[r, pl.ds(q, 16)] * s
                                             + bbuf[qi])

        cslice = pl.ds(cid * _HC, _HC)

        @pl.when(sid < _NS - 1)
        def _():
            pltpu.sync_copy(ybuf, o_ref.at[sl, cslice])

        @pl.when(sid == _NS - 1)
        def _():
            pltpu.sync_copy(ybuf.at[pl.ds(0, _LASTN)],
                            o_ref.at[pl.ds(_LASTY, _LASTN), cslice])

    return k(xwa, xwb, degp, row_t, col_t, zpad, b4)


def _tc_matmul(x, W):
    def body(x_ref, w_ref, a_ref, b_ref):
        xw = jnp.dot(x_ref[...], w_ref[...], preferred_element_type=jnp.float32)
        a_ref[...] = xw[:, :_HC]
        b_ref[...] = xw[:, _HC:]

    return pl.pallas_call(
        body,
        out_shape=(jax.ShapeDtypeStruct((_N, _HC), jnp.float32),
                   jax.ShapeDtypeStruct((_N, _HC), jnp.float32)),
    )(x, W)


def kernel(x, edge_index, W, b):
    row = edge_index[0]
    col = edge_index[1]
    pad = _EPAD - _E
    rowp = jnp.concatenate([row, jnp.zeros((pad,), row.dtype)])
    colp = jnp.concatenate([col, jnp.full((pad,), _N, col.dtype)])
    col_t32 = colp.reshape(_NW, _CHD, _CH)
    row_t16 = rowp.reshape(_NS, _CHH, _CH)
    col_t16 = colp.reshape(_NS, _CHH, _CH)
    z16 = jnp.zeros((_ZS, 16), jnp.float32)
    ones16 = jnp.ones((_CH, 16), jnp.float32)
    zpad = jnp.zeros((_NPAD - _N, _HC), jnp.float32)

    degp = _sc_degree(col_t32, z16, ones16)
    xwa, xwb = _tc_matmul(x, W)
    return _sc_hops(xwa, xwb, degp, row_t16, col_t16, zpad,
                    b.astype(jnp.float32).reshape(4, 16))
